# baseline (speedup 1.0000x reference)
import functools

import jax
import jax.numpy as jnp
from jax import lax
from jax.experimental import pallas as pl
from jax.experimental.pallas import tpu as pltpu
from jax.experimental.pallas import tpu_sc as plsc

N = 1000
C = 81
HW = 28 * 28
LANES = 16
NPAD = 1024

TC_SHEETS = 720
SC_SHEETS = HW - TC_SHEETS
SHEETS_PER_STEP = 48
TC_STEPS = TC_SHEETS // SHEETS_PER_STEP

NC = 2
NS = 16
NW = NC * NS
SC_UNITS = SC_SHEETS * 8
UPW = SC_UNITS // NW
RING = 4


def _tc_body(cid_ref, pred_ref, targ_ref, out_ref):
    step = pl.program_id(0)
    cid = cid_ref[...]
    x = pred_ref[...]
    z = targ_ref[...]

    c_iota = lax.broadcasted_iota(jnp.int32, (1, C, N), 1)
    onehot = cid[:, None, :] == c_iota
    y = jnp.sum(jnp.where(onehot, x, 0.0), axis=1)

    bce = jnp.maximum(y, 0.0) - y * z + jnp.log1p(jnp.exp(-jnp.abs(y)))
    wmask = (cid > 0).astype(jnp.float32)
    step_sum = jnp.sum(bce * wmask).reshape(1, 1)

    @pl.when(step == 0)
    def _():
        out_ref[...] = jnp.zeros_like(out_ref)

    out_ref[0:1, 0:1] += step_sum

    @pl.when(step == TC_STEPS - 1)
    def _():
        out_ref[0:1, 1:2] = jnp.sum(wmask).reshape(1, 1)


def _sc_body(cid_hbm, pred_hbm, targ_hbm, out_hbm,
             cid_v, pbufs, tbufs, stage_v, psems, tsems):
    c = lax.axis_index("c")
    s = lax.axis_index("s")
    wid = s * NC + c
    ubase = wid * UPW

    pltpu.sync_copy(cid_hbm.at[:], cid_v)
    lane = lax.iota(jnp.int32, LANES)

    def _issue(t, r):
        u = ubase + t
        sheet = TC_SHEETS + u // 8
        lt = u % 8
        pltpu.async_copy(
            pred_hbm.at[sheet, :, pl.ds(lt * 128, 128)], pbufs[r], psems[r])
        pltpu.async_copy(
            targ_hbm.at[sheet, pl.ds(lt * 128, 128)], tbufs[r], tsems[r])

    def _compute(t, r, acc):
        lt = (ubase + t) % 8

        def grp_body(g, a):
            cid16 = cid_v[pl.ds(lt * 128 + g * LANES, LANES)]
            x = plsc.load_gather(pbufs[r], [cid16, g * LANES + lane])
            z = tbufs[r][pl.ds(g * LANES, LANES)]
            w = jnp.where(cid16 > 0, 1.0, 0.0).astype(jnp.float32)
            e = jnp.exp(-jnp.abs(x))
            t_ = e / (2.0 + e)
            t2 = t_ * t_
            lg = t_ * (2.0 + t2 * (2.0 / 3.0 + t2 * (2.0 / 5.0 + t2 * (
                2.0 / 7.0 + t2 * (2.0 / 9.0)))))
            bce = jnp.maximum(x, 0.0) - x * z + lg
            return a + w * bce

        return lax.fori_loop(0, 8, grp_body, acc)

    for r in range(RING):
        _issue(r, r)

    def ring_body(k, acc):
        for r in range(RING):
            t = k * RING + r
            pltpu.make_async_copy(
                pred_hbm.at[TC_SHEETS, :, pl.ds(0, 128)],
                pbufs[r], psems[r]).wait()
            pltpu.make_async_copy(
                targ_hbm.at[TC_SHEETS, pl.ds(0, 128)],
                tbufs[r], tsems[r]).wait()
            acc = _compute(t, r, acc)

            @pl.when(t + RING < UPW)
            def _():
                _issue(t + RING, r)
        return acc

    acc = lax.fori_loop(0, UPW // RING, ring_body,
                        jnp.zeros((LANES,), jnp.float32))
    stage_v[...] = acc
    pltpu.sync_copy(stage_v, out_hbm.at[wid])


@functools.partial(
    pl.kernel,
    out_type=jax.ShapeDtypeStruct((NW, LANES), jnp.float32),
    mesh=plsc.VectorSubcoreMesh(core_axis_name="c", subcore_axis_name="s",
                                num_cores=NC, num_subcores=NS),
    compiler_params=pltpu.CompilerParams(needs_layout_passes=False,
                                         use_tc_tiling_on_sc=True),
    scratch_types=[
        pltpu.VMEM((NPAD,), jnp.int32),
        [pltpu.VMEM((C, 128), jnp.float32)] * RING,
        [pltpu.VMEM((128,), jnp.float32)] * RING,
        pltpu.VMEM((LANES,), jnp.float32),
        [pltpu.SemaphoreType.DMA] * RING,
        [pltpu.SemaphoreType.DMA] * RING,
    ],
)
def _sc_partials(cid_hbm, pred_hbm, targ_hbm, out_hbm, *scratch):
    _sc_body(cid_hbm, pred_hbm, targ_hbm, out_hbm, *scratch)


def _fin_body(tc_ref, sc_ref, o_ref):
    total = tc_ref[0, 0] + jnp.sum(sc_ref[...])
    npos = tc_ref[0, 1]
    denom = jnp.maximum(npos, 1.0) * float(HW)
    o_ref[...] = (total / denom).reshape(1, 1)


def kernel(target_masks, target_class_ids, pred_masks):
    cid = target_class_ids.astype(jnp.int32)
    cid2d = cid.reshape(1, N)
    cid_pad = jnp.zeros((NPAD,), jnp.int32).at[:N].set(cid)
    predt = jnp.transpose(pred_masks, (2, 3, 1, 0)).reshape(HW, C, N)
    targt = jnp.transpose(target_masks, (1, 2, 0)).reshape(HW, N)

    tc_out = pl.pallas_call(
        _tc_body,
        grid=(TC_STEPS,),
        in_specs=[
            pl.BlockSpec((1, N), lambda s: (0, 0)),
            pl.BlockSpec((SHEETS_PER_STEP, C, N), lambda s: (s, 0, 0)),
            pl.BlockSpec((SHEETS_PER_STEP, N), lambda s: (s, 0)),
        ],
        out_specs=pl.BlockSpec((1, 2), lambda s: (0, 0)),
        out_shape=jax.ShapeDtypeStruct((1, 2), jnp.float32),
        compiler_params=pltpu.CompilerParams(
            dimension_semantics=("arbitrary",),
            vmem_limit_bytes=100 * 1024 * 1024),
    )(cid2d, predt, targt)

    sc_out = _sc_partials(cid_pad, predt, targt)

    loss = pl.pallas_call(
        _fin_body,
        out_shape=jax.ShapeDtypeStruct((1, 1), jnp.float32),
    )(tc_out, sc_out)
    return loss[0, 0]


# speedup vs baseline: 1.1876x; 1.1876x over previous
import jax
import jax.numpy as jnp
from jax import lax
from jax.experimental import pallas as pl
from jax.experimental.pallas import tpu as pltpu

N = 1000
C = 81
HW = 28 * 28
SHEETS_PER_STEP = 56
STEPS = HW // SHEETS_PER_STEP


def _tc_body(cid_ref, pred_ref, targ_ref, out_ref):
    step = pl.program_id(0)
    cid = cid_ref[...]
    x = pred_ref[...]
    z = targ_ref[...]

    c_iota = lax.broadcasted_iota(jnp.int32, (1, C, N), 1)
    onehot = ((cid[:, None, :] == c_iota) & (c_iota > 0)).astype(jnp.float32)
    y = jnp.sum(x * onehot, axis=1)

    bce = jnp.maximum(y, 0.0) - y * z + jnp.log1p(jnp.exp(-jnp.abs(y)))
    step_sum = jnp.sum(bce).reshape(1, 1)

    @pl.when(step == 0)
    def _():
        out_ref[...] = jnp.zeros_like(out_ref)

    out_ref[0:1, 0:1] += step_sum

    @pl.when(step == STEPS - 1)
    def _():
        npos = jnp.sum((cid > 0).astype(jnp.float32))
        ln2 = jnp.log1p(jnp.exp(-jnp.abs(jnp.float32(0.0))))
        total = out_ref[0, 0] - (float(N) - npos) * float(HW) * ln2
        denom = jnp.maximum(npos, 1.0) * float(HW)
        out_ref[...] = (total / denom).reshape(1, 1)


def kernel(target_masks, target_class_ids, pred_masks):
    cid = target_class_ids.astype(jnp.int32).reshape(1, N)
    predt = jnp.transpose(pred_masks, (2, 3, 1, 0)).reshape(HW, C, N)
    targt = jnp.transpose(target_masks, (1, 2, 0)).reshape(HW, N)
    loss = pl.pallas_call(
        _tc_body,
        grid=(STEPS,),
        in_specs=[
            pl.BlockSpec((1, N), lambda s: (0, 0)),
            pl.BlockSpec((SHEETS_PER_STEP, C, N), lambda s: (s, 0, 0)),
            pl.BlockSpec((SHEETS_PER_STEP, N), lambda s: (s, 0)),
        ],
        out_specs=pl.BlockSpec((1, 1), lambda s: (0, 0)),
        out_shape=jax.ShapeDtypeStruct((1, 1), jnp.float32),
        compiler_params=pltpu.CompilerParams(
            dimension_semantics=("arbitrary",),
            vmem_limit_bytes=100 * 1024 * 1024),
    )(cid, predt, targt)
    return loss[0, 0]
